# Initial kernel scaffold; baseline (speedup 1.0000x reference)
import functools

import jax
import jax.numpy as jnp
from jax import lax
from jax.experimental import pallas as pl
from jax.experimental.pallas import tpu as pltpu
from jax.experimental.pallas import tpu_sc as plsc

N_NODES = 10000
N_EDGES = 320000
D_IN = 128
D_HID = 256
D_OUT = 128

NC = 2
NS = 16
NW = NC * NS
EDGES_PER_W = N_EDGES // NW
CHUNK = 80
NSTEPS = EDGES_PER_W // CHUNK
ROWS_PER_TILE = N_NODES // NS
DEG_W = 16
ZROWS = 125


def _seg_sum_body(with_deg, x_hbm, src_hbm, dst_hbm, *refs):
    if with_deg:
        acc_out, deg_out, src_v, dst_v, rows_v, ones_v, zbuf_v, acc_sh, deg_sh, sem = refs
    else:
        acc_out, src_v, dst_v, rows_v, zbuf_v, acc_sh, sem = refs
        deg_out = deg_sh = ones_v = None

    c = lax.axis_index("c")
    s = lax.axis_index("s")
    wid = s * NC + c

    def zb(i, _):
        r = i // 8
        col = (i % 8) * 16
        zbuf_v[r, pl.ds(col, 16)] = jnp.zeros((16,), jnp.float32)
        return 0
    lax.fori_loop(0, ZROWS * 8, zb, 0)
    if with_deg:
        def ob(i, _):
            ones_v[i, :] = jnp.ones((16,), jnp.float32)
            return 0
        lax.fori_loop(0, CHUNK, ob, 0)

    r0 = s * ROWS_PER_TILE
    for k in range(ROWS_PER_TILE // ZROWS):
        pltpu.sync_copy(zbuf_v, acc_sh.at[pl.ds(r0 + k * ZROWS, ZROWS)])
        if with_deg:
            pltpu.sync_copy(zbuf_v.at[:, pl.ds(0, DEG_W)],
                            deg_sh.at[pl.ds(r0 + k * ZROWS, ZROWS)])
    plsc.subcore_barrier()

    def step(i, _):
        base = pl.multiple_of(wid * EDGES_PER_W + i * CHUNK, CHUNK)
        pltpu.sync_copy(src_hbm.at[pl.ds(base, CHUNK)], src_v)
        pltpu.sync_copy(dst_hbm.at[pl.ds(base, CHUNK)], dst_v)
        pltpu.async_copy(x_hbm.at[src_v], rows_v, sem).wait()
        pltpu.sync_copy(rows_v, acc_sh.at[dst_v], add=True)
        if with_deg:
            pltpu.sync_copy(ones_v, deg_sh.at[dst_v], add=True)
        return 0
    lax.fori_loop(0, NSTEPS, step, 0)
    plsc.subcore_barrier()

    pltpu.sync_copy(acc_sh.at[pl.ds(r0, ROWS_PER_TILE)],
                    acc_out.at[c, pl.ds(r0, ROWS_PER_TILE)])
    if with_deg:
        pltpu.sync_copy(deg_sh.at[pl.ds(r0, ROWS_PER_TILE)],
                        deg_out.at[c, pl.ds(r0, ROWS_PER_TILE)])


def _make_seg_sum(with_deg, d):
    if with_deg:
        out_type = (jax.ShapeDtypeStruct((NC, N_NODES, d), jnp.float32),
                    jax.ShapeDtypeStruct((NC, N_NODES, DEG_W), jnp.float32))
        scratch = [
            pltpu.VMEM((CHUNK,), jnp.int32),
            pltpu.VMEM((CHUNK,), jnp.int32),
            pltpu.VMEM((CHUNK, d), jnp.float32),
            pltpu.VMEM((CHUNK, DEG_W), jnp.float32),
            pltpu.VMEM((ZROWS, d), jnp.float32),
            pltpu.VMEM_SHARED((N_NODES, d), jnp.float32),
            pltpu.VMEM_SHARED((N_NODES, DEG_W), jnp.float32),
            pltpu.SemaphoreType.DMA,
        ]
    else:
        out_type = jax.ShapeDtypeStruct((NC, N_NODES, d), jnp.float32)
        scratch = [
            pltpu.VMEM((CHUNK,), jnp.int32),
            pltpu.VMEM((CHUNK,), jnp.int32),
            pltpu.VMEM((CHUNK, d), jnp.float32),
            pltpu.VMEM((ZROWS, d), jnp.float32),
            pltpu.VMEM_SHARED((N_NODES, d), jnp.float32),
            pltpu.SemaphoreType.DMA,
        ]
    return pl.kernel(
        functools.partial(_seg_sum_body, with_deg),
        out_type=out_type,
        mesh=plsc.VectorSubcoreMesh(core_axis_name="c", subcore_axis_name="s"),
        scratch_types=scratch,
    )


_BLK = 1000
_GRID = N_NODES // _BLK


def _dot(a, b):
    return jnp.dot(a, b, preferred_element_type=jnp.float32,
                   precision=lax.Precision.HIGHEST)


def _mid_body(x_ref, a0_ref, a1_ref, d0_ref, d1_ref, ws1_ref, wn1_ref,
              b1_ref, ws2_ref, wn2_ref, b2_ref, u_ref, p2_ref):
    deg = jnp.maximum(d0_ref[:, 0:1] + d1_ref[:, 0:1], 1.0)
    mean = (a0_ref[...] + a1_ref[...]) / deg
    h = _dot(x_ref[...], ws1_ref[...]) + _dot(mean, wn1_ref[...]) + b1_ref[...]
    h = jnp.maximum(h, 0.0)
    u_ref[...] = _dot(h, ws2_ref[...]) + b2_ref[...]
    p2_ref[...] = _dot(h, wn2_ref[...])


def _fin_body(u_ref, a0_ref, a1_ref, d0_ref, d1_ref, out_ref):
    deg = jnp.maximum(d0_ref[:, 0:1] + d1_ref[:, 0:1], 1.0)
    out_ref[...] = u_ref[...] + (a0_ref[...] + a1_ref[...]) / deg


def _row_spec(d):
    return pl.BlockSpec((_BLK, d), lambda i: (i, 0))


def _full_spec(r, c):
    return pl.BlockSpec((r, c), lambda i: (0, 0))


def kernel(x, edge_index, W_self1, W_neigh1, b1, W_self2, W_neigh2, b2):
    src = edge_index[0].astype(jnp.int32)
    dst = edge_index[1].astype(jnp.int32)

    seg1 = _make_seg_sum(True, D_IN)
    agg1, degp = seg1(x, src, dst)

    mid = pl.pallas_call(
        _mid_body,
        grid=(_GRID,),
        in_specs=[
            _row_spec(D_IN), _row_spec(D_IN), _row_spec(D_IN),
            _row_spec(DEG_W), _row_spec(DEG_W),
            _full_spec(D_IN, D_HID), _full_spec(D_IN, D_HID),
            _full_spec(1, D_HID),
            _full_spec(D_HID, D_OUT), _full_spec(D_HID, D_OUT),
            _full_spec(1, D_OUT),
        ],
        out_specs=[_row_spec(D_OUT), _row_spec(D_OUT)],
        out_shape=[
            jax.ShapeDtypeStruct((N_NODES, D_OUT), jnp.float32),
            jax.ShapeDtypeStruct((N_NODES, D_OUT), jnp.float32),
        ],
    )
    u, p2 = mid(x, agg1[0], agg1[1], degp[0], degp[1],
                W_self1, W_neigh1, b1.reshape(1, D_HID),
                W_self2, W_neigh2, b2.reshape(1, D_OUT))

    seg2 = _make_seg_sum(False, D_OUT)
    agg2 = seg2(p2, src, dst)

    fin = pl.pallas_call(
        _fin_body,
        grid=(_GRID,),
        in_specs=[
            _row_spec(D_OUT), _row_spec(D_OUT), _row_spec(D_OUT),
            _row_spec(DEG_W), _row_spec(DEG_W),
        ],
        out_specs=_row_spec(D_OUT),
        out_shape=jax.ShapeDtypeStruct((N_NODES, D_OUT), jnp.float32),
    )
    return fin(u, agg2[0], agg2[1], degp[0], degp[1])


# speedup vs baseline: 5.7354x; 5.7354x over previous
import functools

import jax
import jax.numpy as jnp
from jax import lax
from jax.experimental import pallas as pl
from jax.experimental.pallas import tpu as pltpu
from jax.experimental.pallas import tpu_sc as plsc

N_NODES = 10000
N_EDGES = 320000
D_IN = 128
D_HID = 256
D_OUT = 128

NC = 2
NS = 16
NW = NC * NS
EDGES_PER_W = N_EDGES // NW
CHUNK = 80
NSTEPS = EDGES_PER_W // CHUNK
N_PAD = 10240
ROWS_PER_TILE = N_PAD // NS
ZROWS = 128


def _seg_sum_body(with_deg, x_hbm, src_hbm, dst_hbm, *refs):
    if with_deg:
        acc_out, deg_out, src_v, dst_v, rows_v, ones_v, zdeg_v, zbuf_v, acc_sh, deg_sh, sem = refs
    else:
        acc_out, src_v, dst_v, rows_v, zbuf_v, acc_sh, sem = refs
        deg_out = deg_sh = ones_v = zdeg_v = None

    c = lax.axis_index("c")
    s = lax.axis_index("s")
    wid = s * NC + c

    def zb(i, _):
        r = i // 8
        col = (i % 8) * 16
        zbuf_v[r, pl.ds(col, 16)] = jnp.zeros((16,), jnp.float32)
        return 0
    lax.fori_loop(0, ZROWS * 8, zb, 0)
    if with_deg:
        def ob(i, _):
            ones_v[pl.ds(i * 16, 16)] = jnp.ones((16,), jnp.float32)
            return 0
        lax.fori_loop(0, CHUNK // 16, ob, 0)

        def zd(i, _):
            zdeg_v[pl.ds(i * 16, 16)] = jnp.zeros((16,), jnp.float32)
            return 0
        lax.fori_loop(0, ROWS_PER_TILE // 16, zd, 0)

    r0 = s * ROWS_PER_TILE
    for k in range(ROWS_PER_TILE // ZROWS):
        pltpu.sync_copy(zbuf_v, acc_sh.at[pl.ds(r0 + k * ZROWS, ZROWS)])
    if with_deg:
        pltpu.sync_copy(zdeg_v, deg_sh.at[pl.ds(r0, ROWS_PER_TILE)])
    plsc.subcore_barrier()

    def step(i, _):
        base = pl.multiple_of(wid * EDGES_PER_W + i * CHUNK, CHUNK)
        pltpu.sync_copy(src_hbm.at[pl.ds(base, CHUNK)], src_v)
        pltpu.sync_copy(dst_hbm.at[pl.ds(base, CHUNK)], dst_v)
        pltpu.async_copy(x_hbm.at[src_v], rows_v, sem).wait()
        pltpu.sync_copy(rows_v, acc_sh.at[dst_v], add=True)
        if with_deg:
            pltpu.sync_copy(ones_v, deg_sh.at[dst_v], add=True)
        return 0
    lax.fori_loop(0, NSTEPS, step, 0)
    plsc.subcore_barrier()

    pltpu.sync_copy(acc_sh.at[pl.ds(r0, ROWS_PER_TILE)],
                    acc_out.at[c, pl.ds(r0, ROWS_PER_TILE)])
    if with_deg:
        pltpu.sync_copy(deg_sh.at[pl.ds(r0, ROWS_PER_TILE)],
                        deg_out.at[pl.ds(c * N_PAD + r0, ROWS_PER_TILE)])


def _make_seg_sum(with_deg, d):
    if with_deg:
        out_type = (jax.ShapeDtypeStruct((NC, N_PAD, d), jnp.float32),
                    jax.ShapeDtypeStruct((NC * N_PAD,), jnp.float32))
        scratch = [
            pltpu.VMEM((CHUNK,), jnp.int32),
            pltpu.VMEM((CHUNK,), jnp.int32),
            pltpu.VMEM((CHUNK, d), jnp.float32),
            pltpu.VMEM((CHUNK,), jnp.float32),
            pltpu.VMEM((ROWS_PER_TILE,), jnp.float32),
            pltpu.VMEM((ZROWS, d), jnp.float32),
            pltpu.VMEM_SHARED((N_PAD, d), jnp.float32),
            pltpu.VMEM_SHARED((N_PAD,), jnp.float32),
            pltpu.SemaphoreType.DMA,
        ]
    else:
        out_type = jax.ShapeDtypeStruct((NC, N_PAD, d), jnp.float32)
        scratch = [
            pltpu.VMEM((CHUNK,), jnp.int32),
            pltpu.VMEM((CHUNK,), jnp.int32),
            pltpu.VMEM((CHUNK, d), jnp.float32),
            pltpu.VMEM((ZROWS, d), jnp.float32),
            pltpu.VMEM_SHARED((N_PAD, d), jnp.float32),
            pltpu.SemaphoreType.DMA,
        ]
    return pl.kernel(
        functools.partial(_seg_sum_body, with_deg),
        out_type=out_type,
        mesh=plsc.VectorSubcoreMesh(core_axis_name="c", subcore_axis_name="s"),
        scratch_types=scratch,
    )


_BLK = 1024
_GRID = N_PAD // _BLK


def _dot(a, b):
    return jnp.dot(a, b, preferred_element_type=jnp.float32,
                   precision=lax.Precision.HIGHEST)


def _mid_body(x_ref, a0_ref, a1_ref, d0_ref, d1_ref, ws1_ref, wn1_ref,
              b1_ref, ws2_ref, wn2_ref, b2_ref, u_ref, p2_ref):
    deg = jnp.maximum(d0_ref[...] + d1_ref[...], 1.0)
    mean = (a0_ref[...] + a1_ref[...]) / deg
    h = _dot(x_ref[...], ws1_ref[...]) + _dot(mean, wn1_ref[...]) + b1_ref[...]
    h = jnp.maximum(h, 0.0)
    u_ref[...] = _dot(h, ws2_ref[...]) + b2_ref[...]
    p2_ref[...] = _dot(h, wn2_ref[...])


def _fin_body(u_ref, a0_ref, a1_ref, d0_ref, d1_ref, out_ref):
    deg = jnp.maximum(d0_ref[...] + d1_ref[...], 1.0)
    out_ref[...] = u_ref[...] + (a0_ref[...] + a1_ref[...]) / deg


def _row_spec(d):
    return pl.BlockSpec((_BLK, d), lambda i: (i, 0))


def _full_spec(r, c):
    return pl.BlockSpec((r, c), lambda i: (0, 0))


def kernel(x, edge_index, W_self1, W_neigh1, b1, W_self2, W_neigh2, b2):
    src = edge_index[0].astype(jnp.int32)
    dst = edge_index[1].astype(jnp.int32)
    x_pad = jnp.pad(x, ((0, N_PAD - N_NODES), (0, 0)))

    seg1 = _make_seg_sum(True, D_IN)
    agg1, deg_flat = seg1(x_pad, src, dst)
    deg2 = deg_flat.reshape(NC, N_PAD)
    d0 = deg2[0][:, None]
    d1 = deg2[1][:, None]

    mid = pl.pallas_call(
        _mid_body,
        grid=(_GRID,),
        in_specs=[
            _row_spec(D_IN), _row_spec(D_IN), _row_spec(D_IN),
            _row_spec(1), _row_spec(1),
            _full_spec(D_IN, D_HID), _full_spec(D_IN, D_HID),
            _full_spec(1, D_HID),
            _full_spec(D_HID, D_OUT), _full_spec(D_HID, D_OUT),
            _full_spec(1, D_OUT),
        ],
        out_specs=[_row_spec(D_OUT), _row_spec(D_OUT)],
        out_shape=[
            jax.ShapeDtypeStruct((N_PAD, D_OUT), jnp.float32),
            jax.ShapeDtypeStruct((N_PAD, D_OUT), jnp.float32),
        ],
    )
    u, p2 = mid(x_pad, agg1[0], agg1[1], d0, d1,
                W_self1, W_neigh1, b1.reshape(1, D_HID),
                W_self2, W_neigh2, b2.reshape(1, D_OUT))

    seg2 = _make_seg_sum(False, D_OUT)
    agg2 = seg2(p2, src, dst)

    fin = pl.pallas_call(
        _fin_body,
        grid=(_GRID,),
        in_specs=[
            _row_spec(D_OUT), _row_spec(D_OUT), _row_spec(D_OUT),
            _row_spec(1), _row_spec(1),
        ],
        out_specs=_row_spec(D_OUT),
        out_shape=jax.ShapeDtypeStruct((N_PAD, D_OUT), jnp.float32),
    )
    out = fin(u, agg2[0], agg2[1], d0, d1)
    return out[:N_NODES]
